# baseline (speedup 1.0000x reference)
import functools

import jax
import jax.numpy as jnp
from jax import lax
from jax.experimental import pallas as pl
from jax.experimental.pallas import tpu as pltpu
from jax.experimental.pallas import tpu_sc as plsc

_B = 16384
_D = 64
_NC = 2
_NS = 16
_NW = _NC * _NS
_BPW = _B // _NW


def _sc_gather_body(ue0, ui0, ie0, ii0, ue1, ui1, ie1, ii1,
                    out_u0, out_i0, out_u1, out_i1,
                    idx_v, rows_v, sem):
  wid = lax.axis_index("s") * _NC + lax.axis_index("c")
  base = wid * _BPW
  pairs = (
      (ue0, ui0, out_u0),
      (ie0, ii0, out_i0),
      (ue1, ui1, out_u1),
      (ie1, ii1, out_i1),
  )
  for table, ids, out in pairs:
    pltpu.sync_copy(ids.at[pl.ds(base, _BPW)], idx_v)
    pltpu.async_copy(table.at[idx_v], rows_v, sem).wait()
    pltpu.sync_copy(rows_v, out.at[pl.ds(base, _BPW)])


@jax.jit
def _sc_gather(user_emb_0, user_ids_0, item_emb_0, item_ids_0,
               user_emb_1, user_ids_1, item_emb_1, item_ids_1):
  mesh = plsc.VectorSubcoreMesh(core_axis_name="c", subcore_axis_name="s")
  row_ty = jax.ShapeDtypeStruct((_B, 2 * _D), jnp.float32)
  fn = pl.kernel(
      _sc_gather_body,
      out_type=(row_ty, row_ty, row_ty, row_ty),
      mesh=mesh,
      scratch_types=[
          pltpu.VMEM((_BPW,), jnp.int32),
          pltpu.VMEM((_BPW, 2 * _D), jnp.float32),
          pltpu.SemaphoreType.DMA,
      ],
      compiler_params=pltpu.CompilerParams(use_tc_tiling_on_sc=True),
  )
  return fn(user_emb_0, user_ids_0, item_emb_0, item_ids_0,
            user_emb_1, user_ids_1, item_emb_1, item_ids_1)


def _xpose_body(tin, tout):
  tout[...] = tin[...].T


@jax.jit
def _tc_xpose(t):
  n = t.shape[1]
  c = 2048
  return pl.pallas_call(
      _xpose_body,
      grid=(pl.cdiv(n, c),),
      in_specs=[pl.BlockSpec((_D, c), lambda i: (0, i))],
      out_specs=pl.BlockSpec((c, _D), lambda i: (i, 0)),
      out_shape=jax.ShapeDtypeStruct((n, _D), jnp.float32),
  )(t)


def _dense_body(u0, i0, u1, i1, pu0, pi0, pu1, pi1, s0, st0, st1, out):
  sa = jnp.concatenate([s0[...], st0[...]], axis=1)
  sb = jnp.concatenate([s0[...], st1[...]], axis=1)

  def half(rows, par):
    l, r = rows[:, :_D], rows[:, _D:]
    return l + (r - l) * par

  ru0 = half(u0[...], pu0[...])
  ri0 = half(i0[...], pi0[...])
  ru1 = half(u1[...], pu1[...])
  ri1 = half(i1[...], pi1[...])
  p0 = jnp.sum(jnp.dot(ru0, sa, preferred_element_type=jnp.float32) * ri0,
               axis=-1)
  p1 = jnp.sum(jnp.dot(ru1, sb, preferred_element_type=jnp.float32) * ri1,
               axis=-1)
  out[...] = jnp.stack([p0, p1], axis=0)


@jax.jit
def _dense(u0, i0, u1, i1, pu0, pi0, pu1, pi1, s0, st0, st1):
  bs = 2048
  grid = (_B // bs,)
  row_spec = pl.BlockSpec((bs, 2 * _D), lambda i: (i, 0))
  par_spec = pl.BlockSpec((bs, 1), lambda i: (i, 0))
  s_spec = pl.BlockSpec((_D, 32), lambda i: (0, 0))
  return pl.pallas_call(
      _dense_body,
      grid=grid,
      in_specs=[row_spec, row_spec, row_spec, row_spec,
                par_spec, par_spec, par_spec, par_spec,
                s_spec, s_spec, s_spec],
      out_specs=pl.BlockSpec((2, bs), lambda i: (0, i)),
      out_shape=jax.ShapeDtypeStruct((2, _B), jnp.float32),
  )(u0, i0, u1, i1, pu0, pi0, pu1, pi1, s0, st0, st1)


def kernel(user_ids_0, item_ids_0, user_ids_1, item_ids_1,
           user_emb_0, user_emb_1, item_emb_0, item_emb_1,
           S0, St_0, St_1):
  tables = [jnp.reshape(t, (-1, 2 * _D))
            for t in (user_emb_0, item_emb_0)]
  tables += [jnp.reshape(_tc_xpose(t.T), (-1, 2 * _D))
             for t in (user_emb_1, item_emb_1)]
  ids = [user_ids_0, item_ids_0, user_ids_1, item_ids_1]
  hi = [i >> 1 for i in ids]
  par = [jnp.reshape((i & 1).astype(jnp.float32), (_B, 1)) for i in ids]
  u0, i0, u1, i1 = _sc_gather(
      tables[0], hi[0], tables[1], hi[1],
      tables[2], hi[2], tables[3], hi[3])
  return _dense(u0, i0, u1, i1, par[0], par[1], par[2], par[3],
                S0, St_0, St_1)


# speedup vs baseline: 1.3001x; 1.3001x over previous
import functools

import jax
import jax.numpy as jnp
from jax import lax
from jax.experimental import pallas as pl
from jax.experimental.pallas import tpu as pltpu
from jax.experimental.pallas import tpu_sc as plsc

_B = 16384
_D = 64
_NC = 2
_NS = 16
_NW = _NC * _NS
_BPW = _B // _NW


def _sc_gather_body(ue0, ui0, ie0, ii0, ue1, ui1, ie1, ii1,
                    out_u0, out_i0, out_u1, out_i1,
                    idx_v, rows_v, sem):
  wid = lax.axis_index("s") * _NC + lax.axis_index("c")
  base = wid * _BPW
  pairs = (
      (ue0, ui0, out_u0),
      (ie0, ii0, out_i0),
      (ue1, ui1, out_u1),
      (ie1, ii1, out_i1),
  )
  for table, ids, out in pairs:
    pltpu.sync_copy(ids.at[pl.ds(base, _BPW)], idx_v)
    pltpu.async_copy(table.at[idx_v], rows_v, sem).wait()
    pltpu.sync_copy(rows_v, out.at[pl.ds(base, _BPW)])


@jax.jit
def _sc_gather(user_emb_0, user_ids_0, item_emb_0, item_ids_0,
               user_emb_1, user_ids_1, item_emb_1, item_ids_1):
  mesh = plsc.VectorSubcoreMesh(core_axis_name="c", subcore_axis_name="s")
  row_ty = jax.ShapeDtypeStruct((_B, _D), jnp.float32)
  fn = pl.kernel(
      _sc_gather_body,
      out_type=(row_ty, row_ty, row_ty, row_ty),
      mesh=mesh,
      scratch_types=[
          pltpu.VMEM((_BPW,), jnp.int32),
          pltpu.VMEM((_BPW, _D), jnp.float32),
          pltpu.SemaphoreType.DMA,
      ],
      compiler_params=pltpu.CompilerParams(use_tc_tiling_on_sc=False),
  )
  return fn(user_emb_0, user_ids_0, item_emb_0, item_ids_0,
            user_emb_1, user_ids_1, item_emb_1, item_ids_1)


def _dense_body(u0, i0, u1, i1, s0, st0, st1, out):
  sa = jnp.concatenate([s0[...], st0[...]], axis=1)
  sb = jnp.concatenate([s0[...], st1[...]], axis=1)
  p0 = jnp.sum(jnp.dot(u0[...], sa, preferred_element_type=jnp.float32)
               * i0[...], axis=-1)
  p1 = jnp.sum(jnp.dot(u1[...], sb, preferred_element_type=jnp.float32)
               * i1[...], axis=-1)
  out[...] = jnp.stack([p0, p1], axis=0)


@jax.jit
def _dense(u0, i0, u1, i1, s0, st0, st1):
  bs = 2048
  grid = (_B // bs,)
  row_spec = pl.BlockSpec((bs, _D), lambda i: (i, 0))
  s_spec = pl.BlockSpec((_D, 32), lambda i: (0, 0))
  return pl.pallas_call(
      _dense_body,
      grid=grid,
      in_specs=[row_spec, row_spec, row_spec, row_spec, s_spec, s_spec, s_spec],
      out_specs=pl.BlockSpec((2, bs), lambda i: (0, i)),
      out_shape=jax.ShapeDtypeStruct((2, _B), jnp.float32),
  )(u0, i0, u1, i1, s0, st0, st1)


def kernel(user_ids_0, item_ids_0, user_ids_1, item_ids_1,
           user_emb_0, user_emb_1, item_emb_0, item_emb_1,
           S0, St_0, St_1):
  u0, i0, u1, i1 = _sc_gather(
      user_emb_0, user_ids_0, item_emb_0, item_ids_0,
      user_emb_1, user_ids_1, item_emb_1, item_ids_1)
  return _dense(u0, i0, u1, i1, S0, St_0, St_1)


# speedup vs baseline: 1.3053x; 1.0040x over previous
import functools

import jax
import jax.numpy as jnp
from jax import lax
from jax.experimental import pallas as pl
from jax.experimental.pallas import tpu as pltpu
from jax.experimental.pallas import tpu_sc as plsc

_B = 16384
_D = 64
_NC = 2
_NS = 16
_NW = _NC * _NS
_BPW = _B // _NW


def _sc_gather_body(table, ids, out, idx_v, rows_v, sem):
  wid = lax.axis_index("s") * _NC + lax.axis_index("c")
  base = wid * _BPW
  pltpu.sync_copy(ids.at[pl.ds(base, _BPW)], idx_v)
  pltpu.async_copy(table.at[idx_v], rows_v, sem).wait()
  pltpu.sync_copy(rows_v, out.at[pl.ds(base, _BPW)])


@jax.jit
def _sc_gather1(table, ids):
  mesh = plsc.VectorSubcoreMesh(core_axis_name="c", subcore_axis_name="s")
  fn = pl.kernel(
      _sc_gather_body,
      out_type=jax.ShapeDtypeStruct((_B, _D), jnp.float32),
      mesh=mesh,
      scratch_types=[
          pltpu.VMEM((_BPW,), jnp.int32),
          pltpu.VMEM((_BPW, _D), jnp.float32),
          pltpu.SemaphoreType.DMA,
      ],
      compiler_params=pltpu.CompilerParams(use_tc_tiling_on_sc=False),
  )
  return fn(table, ids)


def _dense_body(u0, i0, u1, i1, s0, st0, st1, out):
  sa = jnp.concatenate([s0[...], st0[...]], axis=1)
  sb = jnp.concatenate([s0[...], st1[...]], axis=1)
  p0 = jnp.sum(jnp.dot(u0[...], sa, preferred_element_type=jnp.float32)
               * i0[...], axis=-1)
  p1 = jnp.sum(jnp.dot(u1[...], sb, preferred_element_type=jnp.float32)
               * i1[...], axis=-1)
  out[...] = jnp.stack([p0, p1], axis=0)


@jax.jit
def _dense(u0, i0, u1, i1, s0, st0, st1):
  bs = 2048
  grid = (_B // bs,)
  row_spec = pl.BlockSpec((bs, _D), lambda i: (i, 0))
  s_spec = pl.BlockSpec((_D, 32), lambda i: (0, 0))
  return pl.pallas_call(
      _dense_body,
      grid=grid,
      in_specs=[row_spec, row_spec, row_spec, row_spec, s_spec, s_spec, s_spec],
      out_specs=pl.BlockSpec((2, bs), lambda i: (0, i)),
      out_shape=jax.ShapeDtypeStruct((2, _B), jnp.float32),
  )(u0, i0, u1, i1, s0, st0, st1)


def kernel(user_ids_0, item_ids_0, user_ids_1, item_ids_1,
           user_emb_0, user_emb_1, item_emb_0, item_emb_1,
           S0, St_0, St_1):
  u0 = _sc_gather1(user_emb_0, user_ids_0)
  i0 = _sc_gather1(item_emb_0, item_ids_0)
  u1 = _sc_gather1(user_emb_1, user_ids_1)
  i1 = _sc_gather1(item_emb_1, item_ids_1)
  return _dense(u0, i0, u1, i1, S0, St_0, St_1)
